# baseline (speedup 1.0000x reference)
import functools

import jax
import jax.numpy as jnp
from jax import lax
from jax.experimental import pallas as pl
from jax.experimental.pallas import tpu as pltpu
from jax.experimental.pallas import tpu_sc as plsc

_EPS = 1e-8
_MATCH_THRESH = 0.999
_BB = 128
_NSC = 384
_NW = 32

_NT = (((1,), (1,)), ((), ()))


def _tc_kernel(hypo_ref, prem_ref, w_ref, b_ref, out_ref, *, n_pairs, n_h):
    d = hypo_ref.shape[2]
    inv_h = 1.0 / float(n_h)
    ones_row = jnp.ones((1, d), jnp.float32)
    rows = []
    for i in range(n_pairs):
        h = hypo_ref[i]
        p = prem_ref[i]
        ssq_h = lax.dot_general(h * h, ones_row, _NT,
                                preferred_element_type=jnp.float32)
        ssq_p = lax.dot_general(ones_row, p * p, _NT,
                                preferred_element_type=jnp.float32)
        ih = 1.0 / (jnp.sqrt(ssq_h) + _EPS)
        ip = 1.0 / (jnp.sqrt(ssq_p) + _EPS)
        g = lax.dot_general(h, p, _NT,
                            preferred_element_type=jnp.float32)
        c = g * ih * ip
        mx = jnp.max(c, axis=1, keepdims=True)
        s_mx = jnp.sum(mx, axis=0, keepdims=True)
        mn = jnp.min(mx, axis=0, keepdims=True)
        m = jnp.where(c > _MATCH_THRESH, 1.0, 0.0)
        cnt = jnp.sum(jnp.sum(m, axis=1, keepdims=True), axis=0,
                      keepdims=True)
        f0 = (mn > _MATCH_THRESH).astype(jnp.float32)
        f1 = (cnt == float(n_h)).astype(jnp.float32)
        f2 = cnt * inv_h
        f3 = 1.0 - s_mx * inv_h
        f4 = 1.0 - mn
        rows.append(jnp.concatenate([f0, f1, f2, f3, f4], axis=1))
    feats = jnp.concatenate(rows, axis=0)
    out = lax.dot_general(feats, w_ref[:], _NT,
                          preferred_element_type=jnp.float32)
    out_ref[:] = out + b_ref[:]


def _tc_part(hypo, prem, w, b2, n_tc):
    _, H, D = hypo.shape
    P = prem.shape[1]
    bb = _BB
    return pl.pallas_call(
        functools.partial(_tc_kernel, n_pairs=bb, n_h=H),
        grid=(n_tc // bb,),
        in_specs=[
            pl.BlockSpec((bb, H, D), lambda i: (i, 0, 0)),
            pl.BlockSpec((bb, P, D), lambda i: (i, 0, 0)),
            pl.BlockSpec((3, 5), lambda i: (0, 0)),
            pl.BlockSpec((1, 3), lambda i: (0, 0)),
        ],
        out_specs=pl.BlockSpec((bb, 3), lambda i: (i, 0)),
        out_shape=jax.ShapeDtypeStruct((n_tc, 3), jnp.float32),
        compiler_params=pltpu.CompilerParams(
            dimension_semantics=("arbitrary",),
        ),
    )(hypo, prem, w, b2)


def _invnorm16(q):
    s = jnp.full((16,), 17.320508, jnp.float32)
    for _ in range(5):
        s = 0.5 * (s + q / s)
    return 1.0 / (s + _EPS)


_GATHER_DNUMS = lax.GatherDimensionNumbers(
    offset_dims=(), collapsed_slice_dims=(0,), start_index_map=(0,))


def _lane():
    return lax.broadcasted_iota(jnp.int32, (16,), 0)


def _shuffle(x, perm):
    return lax.gather(x, perm.reshape(16, 1), _GATHER_DNUMS,
                      slice_sizes=(1,),
                      mode=lax.GatherScatterMode.PROMISE_IN_BOUNDS)


def _lanesum(x):
    il = _lane()
    for s in (8, 4, 2, 1):
        x = x + _shuffle(x, jnp.bitwise_xor(il, s))
    return x


def _chunk_starts(d):
    starts = list(range(0, d - 15, 16))
    dup = 0
    if d % 16:
        starts.append(d - 16)
        dup = 16 - d % 16
    return starts, dup


def _row_dot(aref, ai, bref, bi, starts, dup, mask_dup):
    acc = jnp.zeros((16,), jnp.float32)
    for n, st in enumerate(starts):
        va = aref[ai, pl.ds(st, 16)]
        vb = bref[bi, pl.ds(st, 16)]
        if dup and n == len(starts) - 1:
            va = jnp.where(mask_dup, 0.0, va)
        acc = acc + va * vb
    return _lanesum(acc)


def _sc_feats_kernel(hypo_hbm, prem_hbm, wexp_hbm, out_hbm, hpad, ppad, fvec,
                     ip_s, wv, sem, *, n_h, n_p, d, base, npw):
    starts, dup = _chunk_starts(d)
    wid = lax.axis_index("c") * 16 + lax.axis_index("s")
    zero16 = jnp.zeros((16,), jnp.float32)
    mask_dup = _lane() < dup
    pltpu.sync_copy(wexp_hbm, wv)
    w0 = wv[pl.ds(0, 16)]
    w1 = wv[pl.ds(16, 16)]
    w2 = wv[pl.ds(32, 16)]

    def pair_body(t, carry_unused):
        pair = base + wid * npw + t
        c1 = pltpu.async_copy(hypo_hbm.at[pair], hpad, sem)
        c2 = pltpu.async_copy(prem_hbm.at[pair], ppad, sem)
        c1.wait()
        c2.wait()

        def pnorm_body(j, carry):
            q = _row_dot(ppad, j, ppad, j, starts, dup, mask_dup)
            ip_s[pl.ds(j * 16, 16)] = _invnorm16(q)
            return carry

        lax.fori_loop(0, n_p, pnorm_body, 0)

        ihs = [_invnorm16(_row_dot(hpad, i, hpad, i, starts, dup, mask_dup))
               for i in range(n_h)]

        cnt = zero16
        s_mx = zero16
        mn = jnp.full((16,), 2.0, jnp.float32)
        nlast = len(starts) - 1
        for ib in range(0, n_h, 2):
            hrows = []
            for i in (ib, ib + 1):
                row = []
                for n, st in enumerate(starts):
                    va = hpad[i, pl.ds(st, 16)]
                    if dup and n == nlast:
                        va = jnp.where(mask_dup, 0.0, va)
                    row.append(va)
                hrows.append(row)
            ih0 = ihs[ib]
            ih1 = ihs[ib + 1]

            def jbody(j, c, _r0=hrows[0], _r1=hrows[1], _ih0=ih0, _ih1=ih1):
                mx0, mx1, cn = c
                a0 = zero16
                a1 = zero16
                for n, st in enumerate(starts):
                    vb = ppad[j, pl.ds(st, 16)]
                    a0 = a0 + _r0[n] * vb
                    a1 = a1 + _r1[n] * vb
                ipj = ip_s[pl.ds(j * 16, 16)]
                c0 = _lanesum(a0) * _ih0 * ipj
                c1 = _lanesum(a1) * _ih1 * ipj
                mx0 = jnp.maximum(mx0, c0)
                mx1 = jnp.maximum(mx1, c1)
                cn = (cn + jnp.where(c0 > _MATCH_THRESH, 1.0, 0.0)
                      + jnp.where(c1 > _MATCH_THRESH, 1.0, 0.0))
                return mx0, mx1, cn

            m2 = jnp.full((16,), -2.0, jnp.float32)
            mx_a, mx_b, cnt = lax.fori_loop(0, n_p, jbody, (m2, m2, cnt))
            s_mx = s_mx + mx_a + mx_b
            mn = jnp.minimum(mn, jnp.minimum(mx_a, mx_b))

        inv_h = 1.0 / float(n_h)
        f0 = jnp.where(mn > _MATCH_THRESH, 1.0, 0.0)
        f1 = jnp.where(cnt == float(n_h), 1.0, 0.0)
        f2 = cnt * inv_h
        f3 = 1.0 - s_mx * inv_h
        f4 = 1.0 - mn
        il = _lane()
        v = jnp.where(il == 0, f0,
            jnp.where(il == 1, f1,
            jnp.where(il == 2, f2,
            jnp.where(il == 3, f3,
            jnp.where(il == 4, f4,
            jnp.where(il == 5, jnp.full((16,), 1.0, jnp.float32),
                      zero16))))))
        o0 = _lanesum(w0 * v)
        o1 = _lanesum(w1 * v)
        o2 = _lanesum(w2 * v)
        ov = jnp.where(il == 0, o0,
             jnp.where(il == 1, o1,
             jnp.where(il == 2, o2, zero16)))
        fvec[pl.ds(0, 16)] = ov
        pltpu.sync_copy(fvec.at[pl.ds(0, 8)],
                        out_hbm.at[pl.ds((wid * npw + t) * 8, 8)])
        return 0

    lax.fori_loop(0, npw, pair_body, 0)


def _sc_part(hypo, prem, wexp, n_sc):
    B, H, D = hypo.shape
    P = prem.shape[1]
    npw = n_sc // _NW
    base = B - n_sc
    mesh = plsc.VectorSubcoreMesh(core_axis_name="c", subcore_axis_name="s")
    kfn = functools.partial(_sc_feats_kernel, n_h=H, n_p=P, d=D,
                           base=base, npw=npw)
    return pl.kernel(
        kfn,
        mesh=mesh,
        out_type=jax.ShapeDtypeStruct((n_sc * 8,), jnp.float32),
        scratch_types=[
            pltpu.VMEM((H, D), jnp.float32),
            pltpu.VMEM((P, D), jnp.float32),
            pltpu.VMEM((16,), jnp.float32),
            pltpu.VMEM((P * 16,), jnp.float32),
            pltpu.VMEM((48,), jnp.float32),
            pltpu.SemaphoreType.DMA,
        ],
    )(hypo, prem, wexp)


@jax.jit
def kernel(hypo, prem, W, b):
    B = hypo.shape[0]
    b2 = b.reshape(1, 3)
    n_sc = _NSC
    n_tc = B - n_sc
    wexp = jnp.pad(jnp.concatenate([W, b.reshape(3, 1)], axis=1),
                   ((0, 0), (0, 10))).reshape(48)
    out_sc = _sc_part(hypo, prem, wexp, n_sc).reshape(n_sc, 8)[:, 0:3]
    out_tc = _tc_part(hypo, prem, W, b2, n_tc)
    return jnp.concatenate([out_tc, out_sc], axis=0)


# speedup vs baseline: 1.1065x; 1.1065x over previous
import functools

import jax
import jax.numpy as jnp
from jax import lax
from jax.experimental import pallas as pl
from jax.experimental.pallas import tpu as pltpu
from jax.experimental.pallas import tpu_sc as plsc

_EPS = 1e-8
_MATCH_THRESH = 0.999
_BB = 128
_NSC = 256
_NW = 32

_NT = (((1,), (1,)), ((), ()))


def _tc_kernel(hypo_ref, prem_ref, w_ref, b_ref, out_ref, *, n_pairs, n_h):
    d = hypo_ref.shape[2]
    inv_h = 1.0 / float(n_h)
    ones_row = jnp.ones((1, d), jnp.float32)
    rows = []
    for i in range(n_pairs):
        h = hypo_ref[i]
        p = prem_ref[i]
        ssq_h = lax.dot_general(h * h, ones_row, _NT,
                                preferred_element_type=jnp.float32)
        ssq_p = lax.dot_general(ones_row, p * p, _NT,
                                preferred_element_type=jnp.float32)
        ih = 1.0 / (jnp.sqrt(ssq_h) + _EPS)
        ip = 1.0 / (jnp.sqrt(ssq_p) + _EPS)
        g = lax.dot_general(h, p, _NT,
                            preferred_element_type=jnp.float32)
        c = g * ih * ip
        mx = jnp.max(c, axis=1, keepdims=True)
        s_mx = jnp.sum(mx, axis=0, keepdims=True)
        mn = jnp.min(mx, axis=0, keepdims=True)
        m = jnp.where(c > _MATCH_THRESH, 1.0, 0.0)
        cnt = jnp.sum(jnp.sum(m, axis=1, keepdims=True), axis=0,
                      keepdims=True)
        f0 = (mn > _MATCH_THRESH).astype(jnp.float32)
        f1 = (cnt == float(n_h)).astype(jnp.float32)
        f2 = cnt * inv_h
        f3 = 1.0 - s_mx * inv_h
        f4 = 1.0 - mn
        rows.append(jnp.concatenate([f0, f1, f2, f3, f4], axis=1))
    feats = jnp.concatenate(rows, axis=0)
    out = lax.dot_general(feats, w_ref[:], _NT,
                          preferred_element_type=jnp.float32)
    out_ref[:] = out + b_ref[:]


def _tc_part(hypo, prem, w, b2, n_tc):
    _, H, D = hypo.shape
    P = prem.shape[1]
    bb = _BB
    return pl.pallas_call(
        functools.partial(_tc_kernel, n_pairs=bb, n_h=H),
        grid=(n_tc // bb,),
        in_specs=[
            pl.BlockSpec((bb, H, D), lambda i: (i, 0, 0)),
            pl.BlockSpec((bb, P, D), lambda i: (i, 0, 0)),
            pl.BlockSpec((3, 5), lambda i: (0, 0)),
            pl.BlockSpec((1, 3), lambda i: (0, 0)),
        ],
        out_specs=pl.BlockSpec((bb, 3), lambda i: (i, 0)),
        out_shape=jax.ShapeDtypeStruct((n_tc, 3), jnp.float32),
        compiler_params=pltpu.CompilerParams(
            dimension_semantics=("arbitrary",),
        ),
    )(hypo, prem, w, b2)


def _invnorm16(q):
    s = jnp.full((16,), 17.320508, jnp.float32)
    for _ in range(5):
        s = 0.5 * (s + q / s)
    return 1.0 / (s + _EPS)


_GATHER_DNUMS = lax.GatherDimensionNumbers(
    offset_dims=(), collapsed_slice_dims=(0,), start_index_map=(0,))


def _lane():
    return lax.broadcasted_iota(jnp.int32, (16,), 0)


def _shuffle(x, perm):
    return lax.gather(x, perm.reshape(16, 1), _GATHER_DNUMS,
                      slice_sizes=(1,),
                      mode=lax.GatherScatterMode.PROMISE_IN_BOUNDS)


def _lanesum(x):
    il = _lane()
    for s in (8, 4, 2, 1):
        x = x + _shuffle(x, jnp.bitwise_xor(il, s))
    return x


def _chunk_starts(d):
    starts = list(range(0, d - 15, 16))
    dup = 0
    if d % 16:
        starts.append(d - 16)
        dup = 16 - d % 16
    return starts, dup


def _row_dot(aref, ai, bref, bi, starts, dup, mask_dup):
    acc = jnp.zeros((16,), jnp.float32)
    for n, st in enumerate(starts):
        va = aref[ai, pl.ds(st, 16)]
        vb = bref[bi, pl.ds(st, 16)]
        if dup and n == len(starts) - 1:
            va = jnp.where(mask_dup, 0.0, va)
        acc = acc + va * vb
    return _lanesum(acc)


def _sc_feats_kernel(hypo_hbm, prem_hbm, wexp_hbm, out_hbm, hpad, ppad, fvec,
                     ip_s, wv, sem, *, n_h, n_p, d, base, npw):
    starts, dup = _chunk_starts(d)
    wid = lax.axis_index("c") * 16 + lax.axis_index("s")
    zero16 = jnp.zeros((16,), jnp.float32)
    mask_dup = _lane() < dup
    pltpu.sync_copy(wexp_hbm, wv)
    w0 = wv[pl.ds(0, 16)]
    w1 = wv[pl.ds(16, 16)]
    w2 = wv[pl.ds(32, 16)]

    def pair_body(t, carry_unused):
        pair = base + wid * npw + t
        c1 = pltpu.async_copy(hypo_hbm.at[pair], hpad, sem)
        c2 = pltpu.async_copy(prem_hbm.at[pair], ppad, sem)
        c1.wait()
        c2.wait()

        def pnorm_body(j, carry):
            q = _row_dot(ppad, j, ppad, j, starts, dup, mask_dup)
            ip_s[pl.ds(j * 16, 16)] = _invnorm16(q)
            return carry

        lax.fori_loop(0, n_p, pnorm_body, 0)

        ihs = [_invnorm16(_row_dot(hpad, i, hpad, i, starts, dup, mask_dup))
               for i in range(n_h)]

        cnt = zero16
        s_mx = zero16
        mn = jnp.full((16,), 2.0, jnp.float32)
        nlast = len(starts) - 1
        for ib in range(0, n_h, 2):
            hrows = []
            for i in (ib, ib + 1):
                row = []
                for n, st in enumerate(starts):
                    va = hpad[i, pl.ds(st, 16)]
                    if dup and n == nlast:
                        va = jnp.where(mask_dup, 0.0, va)
                    row.append(va)
                hrows.append(row)
            ih0 = ihs[ib]
            ih1 = ihs[ib + 1]

            def jbody(j, c, _r0=hrows[0], _r1=hrows[1], _ih0=ih0, _ih1=ih1):
                mx0, mx1, cn = c
                a0 = zero16
                a1 = zero16
                for n, st in enumerate(starts):
                    vb = ppad[j, pl.ds(st, 16)]
                    a0 = a0 + _r0[n] * vb
                    a1 = a1 + _r1[n] * vb
                ipj = ip_s[pl.ds(j * 16, 16)]
                c0 = _lanesum(a0) * _ih0 * ipj
                c1 = _lanesum(a1) * _ih1 * ipj
                mx0 = jnp.maximum(mx0, c0)
                mx1 = jnp.maximum(mx1, c1)
                cn = (cn + jnp.where(c0 > _MATCH_THRESH, 1.0, 0.0)
                      + jnp.where(c1 > _MATCH_THRESH, 1.0, 0.0))
                return mx0, mx1, cn

            m2 = jnp.full((16,), -2.0, jnp.float32)
            mx_a, mx_b, cnt = lax.fori_loop(0, n_p, jbody, (m2, m2, cnt))
            s_mx = s_mx + mx_a + mx_b
            mn = jnp.minimum(mn, jnp.minimum(mx_a, mx_b))

        inv_h = 1.0 / float(n_h)
        f0 = jnp.where(mn > _MATCH_THRESH, 1.0, 0.0)
        f1 = jnp.where(cnt == float(n_h), 1.0, 0.0)
        f2 = cnt * inv_h
        f3 = 1.0 - s_mx * inv_h
        f4 = 1.0 - mn
        il = _lane()
        v = jnp.where(il == 0, f0,
            jnp.where(il == 1, f1,
            jnp.where(il == 2, f2,
            jnp.where(il == 3, f3,
            jnp.where(il == 4, f4,
            jnp.where(il == 5, jnp.full((16,), 1.0, jnp.float32),
                      zero16))))))
        o0 = _lanesum(w0 * v)
        o1 = _lanesum(w1 * v)
        o2 = _lanesum(w2 * v)
        ov = jnp.where(il == 0, o0,
             jnp.where(il == 1, o1,
             jnp.where(il == 2, o2, zero16)))
        fvec[pl.ds(0, 16)] = ov
        pltpu.sync_copy(fvec.at[pl.ds(0, 8)],
                        out_hbm.at[pl.ds((wid * npw + t) * 8, 8)])
        return 0

    lax.fori_loop(0, npw, pair_body, 0)


def _sc_part(hypo, prem, wexp, n_sc):
    B, H, D = hypo.shape
    P = prem.shape[1]
    npw = n_sc // _NW
    base = B - n_sc
    mesh = plsc.VectorSubcoreMesh(core_axis_name="c", subcore_axis_name="s")
    kfn = functools.partial(_sc_feats_kernel, n_h=H, n_p=P, d=D,
                           base=base, npw=npw)
    return pl.kernel(
        kfn,
        mesh=mesh,
        out_type=jax.ShapeDtypeStruct((n_sc * 8,), jnp.float32),
        scratch_types=[
            pltpu.VMEM((H, D), jnp.float32),
            pltpu.VMEM((P, D), jnp.float32),
            pltpu.VMEM((16,), jnp.float32),
            pltpu.VMEM((P * 16,), jnp.float32),
            pltpu.VMEM((48,), jnp.float32),
            pltpu.SemaphoreType.DMA,
        ],
    )(hypo, prem, wexp)


@jax.jit
def kernel(hypo, prem, W, b):
    B = hypo.shape[0]
    b2 = b.reshape(1, 3)
    n_sc = _NSC
    n_tc = B - n_sc
    wexp = jnp.pad(jnp.concatenate([W, b.reshape(3, 1)], axis=1),
                   ((0, 0), (0, 10))).reshape(48)
    out_sc = _sc_part(hypo, prem, wexp, n_sc).reshape(n_sc, 8)[:, 0:3]
    out_tc = _tc_part(hypo, prem, W, b2, n_tc)
    return jnp.concatenate([out_tc, out_sc], axis=0)
